# Initial kernel scaffold; baseline (speedup 1.0000x reference)
import jax
import jax.numpy as jnp
from jax.experimental import pallas as pl


def kernel(tokens, seq_lengths, embed_table):
    raise NotImplementedError("write your pallas kernel here")


# speedup vs baseline: 29.9167x; 29.9167x over previous
import functools

import jax
import jax.numpy as jnp
from jax import lax
from jax.experimental import pallas as pl
from jax.experimental.pallas import tpu as pltpu
from jax.experimental.pallas import tpu_sc as plsc

B = 16
L = 4096
VOCAB = 32000
DIM = 1024

NUM_CORES = 2
NUM_SUBCORES = 16
LANES = 16
NUM_WORKERS = NUM_CORES * NUM_SUBCORES
TOK_PER_WORKER = (B * L) // NUM_WORKERS
STEPS = TOK_PER_WORKER // LANES
HALF_L = L // 2

VROWS = VOCAB // 128
VBLK = 25


def _rowsum_body(x_ref, o_ref):
    o_ref[0] = jnp.sum(x_ref[:], axis=2)


def _row_sums(embed_table):
    return pl.pallas_call(
        _rowsum_body,
        grid=(VROWS // VBLK,),
        in_specs=[pl.BlockSpec((VBLK, 128, DIM), lambda i: (i, 0, 0))],
        out_specs=pl.BlockSpec((1, VBLK, 128), lambda i: (i, 0, 0)),
        out_shape=jax.ShapeDtypeStruct((VROWS // VBLK, VBLK, 128), jnp.float32),
    )(embed_table.reshape(VROWS, 128, DIM))


_SC_MESH = plsc.VectorSubcoreMesh(
    core_axis_name="c", subcore_axis_name="s",
    num_cores=NUM_CORES, num_subcores=NUM_SUBCORES,
)


@functools.partial(
    pl.kernel,
    out_type=jax.ShapeDtypeStruct((NUM_WORKERS, LANES), jnp.float32),
    mesh=_SC_MESH,
    compiler_params=pltpu.CompilerParams(needs_layout_passes=False),
    scratch_types=[
        pltpu.VMEM((VOCAB,), jnp.float32),
        pltpu.VMEM((TOK_PER_WORKER,), jnp.int32),
        pltpu.VMEM((LANES,), jnp.int32),
        pltpu.VMEM((LANES,), jnp.float32),
    ],
)
def _sc_gather_sum(rs_hbm, tok_hbm, lim_hbm, out_hbm, rs_v, tok_v, lim_v, acc_v):
    wid = lax.axis_index("s") * NUM_CORES + lax.axis_index("c")
    pltpu.sync_copy(rs_hbm, rs_v)
    pltpu.sync_copy(tok_hbm.at[pl.ds(wid * TOK_PER_WORKER, TOK_PER_WORKER)], tok_v)
    pltpu.sync_copy(lim_hbm.at[wid], lim_v)

    limit = lim_v[...]
    lane_ids = lax.iota(jnp.int32, LANES)

    def body(i, acc):
        idx = tok_v[pl.ds(i * LANES, LANES)]
        pos = i * LANES + lane_ids
        idx = jnp.where(pos < limit, idx, 0)
        return acc + plsc.load_gather(rs_v, [idx])

    acc_v[...] = lax.fori_loop(0, STEPS, body, jnp.zeros((LANES,), jnp.float32))
    pltpu.sync_copy(acc_v, out_hbm.at[wid])


def kernel(tokens, seq_lengths, embed_table):
    row_sums = _row_sums(embed_table).reshape(VOCAB)
    w = jnp.arange(NUM_WORKERS)
    limits = seq_lengths[w // 2] - (w % 2) * HALF_L
    limits = jnp.broadcast_to(limits[:, None], (NUM_WORKERS, LANES))
    partials = _sc_gather_sum(row_sums, tokens.reshape(-1), limits)
    return jnp.sum(partials)


# speedup vs baseline: 31.0613x; 1.0383x over previous
import functools

import jax
import jax.numpy as jnp
from jax import lax
from jax.experimental import pallas as pl
from jax.experimental.pallas import tpu as pltpu
from jax.experimental.pallas import tpu_sc as plsc

B = 16
L = 4096
VOCAB = 32000
DIM = 1024

NUM_CORES = 2
NUM_SUBCORES = 16
LANES = 16
NUM_WORKERS = NUM_CORES * NUM_SUBCORES
TOK_PER_WORKER = (B * L) // NUM_WORKERS
STEPS = TOK_PER_WORKER // LANES
HALF_L = L // 2

VROWS = VOCAB // 128
VBLK = 10


def _rowsum_body(x_ref, o_ref):
    o_ref[0] = jnp.sum(x_ref[:], axis=2)


def _row_sums(embed_table):
    return pl.pallas_call(
        _rowsum_body,
        grid=(VROWS // VBLK,),
        in_specs=[pl.BlockSpec((VBLK, 128, DIM), lambda i: (i, 0, 0))],
        out_specs=pl.BlockSpec((1, VBLK, 128), lambda i: (i, 0, 0)),
        out_shape=jax.ShapeDtypeStruct((VROWS // VBLK, VBLK, 128), jnp.float32),
    )(embed_table.reshape(VROWS, 128, DIM))


_SC_MESH = plsc.VectorSubcoreMesh(
    core_axis_name="c", subcore_axis_name="s",
    num_cores=NUM_CORES, num_subcores=NUM_SUBCORES,
)


@functools.partial(
    pl.kernel,
    out_type=jax.ShapeDtypeStruct((NUM_WORKERS, LANES), jnp.float32),
    mesh=_SC_MESH,
    compiler_params=pltpu.CompilerParams(needs_layout_passes=False),
    scratch_types=[
        pltpu.VMEM((VOCAB,), jnp.float32),
        pltpu.VMEM((TOK_PER_WORKER,), jnp.int32),
        pltpu.VMEM((LANES,), jnp.int32),
        pltpu.VMEM((LANES,), jnp.float32),
    ],
)
def _sc_gather_sum(rs_hbm, tok_hbm, lim_hbm, out_hbm, rs_v, tok_v, lim_v, acc_v):
    wid = lax.axis_index("s") * NUM_CORES + lax.axis_index("c")
    pltpu.sync_copy(rs_hbm, rs_v)
    pltpu.sync_copy(tok_hbm.at[pl.ds(wid * TOK_PER_WORKER, TOK_PER_WORKER)], tok_v)
    pltpu.sync_copy(lim_hbm.at[wid], lim_v)

    limit = lim_v[...]
    lane_ids = lax.iota(jnp.int32, LANES)

    def body(i, acc):
        idx = tok_v[pl.ds(i * LANES, LANES)]
        pos = i * LANES + lane_ids
        idx = jnp.where(pos < limit, idx, 0)
        return acc + plsc.load_gather(rs_v, [idx])

    acc_v[...] = lax.fori_loop(0, STEPS, body, jnp.zeros((LANES,), jnp.float32))
    pltpu.sync_copy(acc_v, out_hbm.at[wid])


def kernel(tokens, seq_lengths, embed_table):
    row_sums = _row_sums(embed_table).reshape(VOCAB)
    w = jnp.arange(NUM_WORKERS)
    limits = seq_lengths[w // 2] - (w % 2) * HALF_L
    limits = jnp.broadcast_to(limits[:, None], (NUM_WORKERS, LANES))
    partials = _sc_gather_sum(row_sums, tokens.reshape(-1), limits)
    return jnp.sum(partials)


# speedup vs baseline: 31.5989x; 1.0173x over previous
import functools

import jax
import jax.numpy as jnp
from jax import lax
from jax.experimental import pallas as pl
from jax.experimental.pallas import tpu as pltpu
from jax.experimental.pallas import tpu_sc as plsc

B = 16
L = 4096
VOCAB = 32000
DIM = 1024

NUM_CORES = 2
NUM_SUBCORES = 16
LANES = 16
NUM_WORKERS = NUM_CORES * NUM_SUBCORES
TOK_PER_WORKER = (B * L) // NUM_WORKERS
STEPS = TOK_PER_WORKER // LANES
HALF_L = L // 2

VROWS = VOCAB // 128
VBLK = 10


def _rowsum_body(x_ref, o_ref):
    o_ref[0] = jnp.sum(x_ref[:], axis=2)


def _row_sums(embed_table):
    return pl.pallas_call(
        _rowsum_body,
        grid=(VROWS // VBLK,),
        in_specs=[pl.BlockSpec((VBLK, 128, DIM), lambda i: (i, 0, 0))],
        out_specs=pl.BlockSpec((1, VBLK, 128), lambda i: (i, 0, 0)),
        out_shape=jax.ShapeDtypeStruct((VROWS // VBLK, VBLK, 128), jnp.float32),
    )(embed_table.reshape(VROWS, 128, DIM))


_SC_MESH = plsc.VectorSubcoreMesh(
    core_axis_name="c", subcore_axis_name="s",
    num_cores=NUM_CORES, num_subcores=NUM_SUBCORES,
)


@functools.partial(
    pl.kernel,
    out_type=jax.ShapeDtypeStruct((NUM_WORKERS, LANES), jnp.float32),
    mesh=_SC_MESH,
    compiler_params=pltpu.CompilerParams(needs_layout_passes=False),
    scratch_types=[
        pltpu.VMEM((VOCAB,), jnp.float32),
        pltpu.VMEM((TOK_PER_WORKER,), jnp.int32),
        pltpu.VMEM((B,), jnp.int32),
        pltpu.VMEM((LANES,), jnp.float32),
    ],
)
def _sc_gather_sum(rs_hbm, tok_hbm, sl_hbm, out_hbm, rs_v, tok_v, sl_v, acc_v):
    wid = lax.axis_index("s") * NUM_CORES + lax.axis_index("c")
    pltpu.sync_copy(rs_hbm, rs_v)
    pltpu.sync_copy(tok_hbm.at[pl.ds(wid * TOK_PER_WORKER, TOK_PER_WORKER)], tok_v)
    pltpu.sync_copy(sl_hbm, sl_v)

    limit = plsc.load_gather(
        sl_v, [jnp.full((LANES,), wid // 2, jnp.int32)]
    ) - (wid % 2) * HALF_L
    lane_ids = lax.iota(jnp.int32, LANES)

    def body(i, acc):
        idx = tok_v[pl.ds(i * LANES, LANES)]
        pos = i * LANES + lane_ids
        idx = jnp.where(pos < limit, idx, 0)
        return acc + plsc.load_gather(rs_v, [idx])

    acc_v[...] = lax.fori_loop(
        0, STEPS, body, jnp.zeros((LANES,), jnp.float32), unroll=4
    )
    pltpu.sync_copy(acc_v, out_hbm.at[wid])


def kernel(tokens, seq_lengths, embed_table):
    row_sums = _row_sums(embed_table).reshape(VOCAB)
    partials = _sc_gather_sum(row_sums, tokens.reshape(-1), seq_lengths)
    return jnp.sum(partials)


# speedup vs baseline: 31.6610x; 1.0020x over previous
import functools

import jax
import jax.numpy as jnp
from jax import lax
from jax.experimental import pallas as pl
from jax.experimental.pallas import tpu as pltpu
from jax.experimental.pallas import tpu_sc as plsc

B = 16
L = 4096
VOCAB = 32000
DIM = 1024

NUM_CORES = 2
NUM_SUBCORES = 16
LANES = 16
NUM_WORKERS = NUM_CORES * NUM_SUBCORES
TOK_PER_WORKER = L
STEPS = TOK_PER_WORKER // LANES

VROWS = VOCAB // 128
VBLK = 50


def _rowsum_body(x_ref, o_ref):
    o_ref[0] = jnp.sum(x_ref[:], axis=2)


def _row_sums(embed_table):
    return pl.pallas_call(
        _rowsum_body,
        grid=(VROWS // VBLK,),
        in_specs=[pl.BlockSpec((VBLK, 128, DIM), lambda i: (i, 0, 0))],
        out_specs=pl.BlockSpec((1, VBLK, 128), lambda i: (i, 0, 0)),
        out_shape=jax.ShapeDtypeStruct((VROWS // VBLK, VBLK, 128), jnp.float32),
    )(embed_table.reshape(VROWS, 128, DIM))


_SC_MESH = plsc.VectorSubcoreMesh(
    core_axis_name="c", subcore_axis_name="s",
    num_cores=NUM_CORES, num_subcores=NUM_SUBCORES,
)


@functools.partial(
    pl.kernel,
    out_type=jax.ShapeDtypeStruct((B, LANES), jnp.float32),
    mesh=_SC_MESH,
    compiler_params=pltpu.CompilerParams(needs_layout_passes=False),
    scratch_types=[
        pltpu.VMEM((VOCAB,), jnp.float32),
        pltpu.VMEM((TOK_PER_WORKER,), jnp.int32),
        pltpu.VMEM((B,), jnp.int32),
        pltpu.VMEM((LANES,), jnp.float32),
    ],
)
def _sc_gather_sum(rs_hbm, tok_hbm, sl_hbm, out_hbm, rs_v, tok_v, sl_v, acc_v):
    wid = lax.axis_index("s") * NUM_CORES + lax.axis_index("c")

    @pl.when(wid < B)
    def _():
        pltpu.sync_copy(rs_hbm, rs_v)
        pltpu.sync_copy(tok_hbm.at[pl.ds(wid * TOK_PER_WORKER, TOK_PER_WORKER)], tok_v)
        pltpu.sync_copy(sl_hbm, sl_v)

        limit = plsc.load_gather(sl_v, [jnp.full((LANES,), wid, jnp.int32)])
        lane_ids = lax.iota(jnp.int32, LANES)

        def body(i, acc):
            idx = tok_v[pl.ds(i * LANES, LANES)]
            pos = i * LANES + lane_ids
            idx = jnp.where(pos < limit, idx, 0)
            return acc + plsc.load_gather(rs_v, [idx])

        acc_v[...] = lax.fori_loop(
            0, STEPS, body, jnp.zeros((LANES,), jnp.float32), unroll=8
        )
        pltpu.sync_copy(acc_v, out_hbm.at[wid])


def kernel(tokens, seq_lengths, embed_table):
    row_sums = _row_sums(embed_table).reshape(VOCAB)
    partials = _sc_gather_sum(row_sums, tokens.reshape(-1), seq_lengths)
    return jnp.sum(partials)


# speedup vs baseline: 32.3891x; 1.0230x over previous
import functools

import jax
import jax.numpy as jnp
from jax import lax
from jax.experimental import pallas as pl
from jax.experimental.pallas import tpu as pltpu
from jax.experimental.pallas import tpu_sc as plsc

B = 16
L = 4096
VOCAB = 32000
DIM = 1024

NUM_CORES = 2
NUM_SUBCORES = 16
LANES = 16
NUM_WORKERS = NUM_CORES * NUM_SUBCORES
TOK_PER_WORKER = L
STEPS = TOK_PER_WORKER // LANES

VROWS = VOCAB // 128
VBLK = 50


def _rowsum_body(x_ref, o_ref):
    o_ref[0] = jnp.sum(x_ref[:], axis=2)


def _row_sums(embed_table):
    return pl.pallas_call(
        _rowsum_body,
        grid=(VROWS // VBLK,),
        in_specs=[pl.BlockSpec((VBLK, 128, DIM), lambda i: (i, 0, 0))],
        out_specs=pl.BlockSpec((1, VBLK, 128), lambda i: (i, 0, 0)),
        out_shape=jax.ShapeDtypeStruct((VROWS // VBLK, VBLK, 128), jnp.float32),
    )(embed_table.reshape(VROWS, 128, DIM))


_SC_MESH = plsc.VectorSubcoreMesh(
    core_axis_name="c", subcore_axis_name="s",
    num_cores=1, num_subcores=NUM_SUBCORES,
)


@functools.partial(
    pl.kernel,
    out_type=jax.ShapeDtypeStruct((B, LANES), jnp.float32),
    mesh=_SC_MESH,
    compiler_params=pltpu.CompilerParams(needs_layout_passes=False),
    scratch_types=[
        pltpu.VMEM((VOCAB,), jnp.float32),
        pltpu.VMEM((TOK_PER_WORKER,), jnp.int32),
        pltpu.VMEM((B,), jnp.int32),
        pltpu.VMEM((LANES,), jnp.float32),
    ],
)
def _sc_gather_sum(rs_hbm, tok_hbm, sl_hbm, out_hbm, rs_v, tok_v, sl_v, acc_v):
    wid = lax.axis_index("s")

    @pl.when(wid < B)
    def _():
        pltpu.sync_copy(rs_hbm, rs_v)
        pltpu.sync_copy(tok_hbm.at[pl.ds(wid * TOK_PER_WORKER, TOK_PER_WORKER)], tok_v)
        pltpu.sync_copy(sl_hbm, sl_v)

        limit = plsc.load_gather(sl_v, [jnp.full((LANES,), wid, jnp.int32)])
        lane_ids = lax.iota(jnp.int32, LANES)

        def body(i, acc):
            idx = tok_v[pl.ds(i * LANES, LANES)]
            pos = i * LANES + lane_ids
            idx = jnp.where(pos < limit, idx, 0)
            return acc + plsc.load_gather(rs_v, [idx])

        acc_v[...] = lax.fori_loop(
            0, STEPS, body, jnp.zeros((LANES,), jnp.float32), unroll=8
        )
        pltpu.sync_copy(acc_v, out_hbm.at[wid])


def kernel(tokens, seq_lengths, embed_table):
    row_sums = _row_sums(embed_table).reshape(VOCAB)
    partials = _sc_gather_sum(row_sums, tokens.reshape(-1), seq_lengths)
    return jnp.sum(partials)


# speedup vs baseline: 32.4781x; 1.0027x over previous
import functools

import jax
import jax.numpy as jnp
from jax import lax
from jax.experimental import pallas as pl
from jax.experimental.pallas import tpu as pltpu
from jax.experimental.pallas import tpu_sc as plsc

B = 16
L = 4096
VOCAB = 32000
DIM = 1024

NUM_CORES = 2
NUM_SUBCORES = 16
LANES = 16
NUM_WORKERS = NUM_CORES * NUM_SUBCORES
TOK_PER_WORKER = L
STEPS = TOK_PER_WORKER // LANES

VROWS = VOCAB // 128
VBLK = 50


def _rowsum_body(x_ref, o_ref):
    o_ref[0] = jnp.sum(x_ref[:], axis=2)


def _row_sums(embed_table):
    return pl.pallas_call(
        _rowsum_body,
        grid=(VROWS // VBLK,),
        in_specs=[pl.BlockSpec((VBLK, 128, DIM), lambda i: (i, 0, 0))],
        out_specs=pl.BlockSpec((1, VBLK, 128), lambda i: (i, 0, 0)),
        out_shape=jax.ShapeDtypeStruct((VROWS // VBLK, VBLK, 128), jnp.float32),
    )(embed_table.reshape(VROWS, 128, DIM))


_SC_MESH = plsc.VectorSubcoreMesh(
    core_axis_name="c", subcore_axis_name="s",
    num_cores=1, num_subcores=NUM_SUBCORES,
)


@functools.partial(
    pl.kernel,
    out_type=jax.ShapeDtypeStruct((1,), jnp.float32),
    mesh=_SC_MESH,
    compiler_params=pltpu.CompilerParams(needs_layout_passes=False),
    scratch_types=[
        pltpu.VMEM((VOCAB,), jnp.float32),
        pltpu.VMEM((TOK_PER_WORKER,), jnp.int32),
        pltpu.VMEM((B,), jnp.int32),
        pltpu.VMEM((LANES,), jnp.float32),
        pltpu.VMEM((B, LANES), jnp.float32),
        pltpu.VMEM_SHARED((B, LANES), jnp.float32),
    ],
)
def _sc_gather_sum(rs_hbm, tok_hbm, sl_hbm, out_hbm,
                   rs_v, tok_v, sl_v, acc_v, red_v, shared):
    wid = lax.axis_index("s")
    pltpu.sync_copy(rs_hbm, rs_v)
    pltpu.sync_copy(tok_hbm.at[pl.ds(wid * TOK_PER_WORKER, TOK_PER_WORKER)], tok_v)
    pltpu.sync_copy(sl_hbm, sl_v)

    limit = plsc.load_gather(sl_v, [jnp.full((LANES,), wid, jnp.int32)])
    lane_ids = lax.iota(jnp.int32, LANES)

    def body(i, acc):
        idx = tok_v[pl.ds(i * LANES, LANES)]
        pos = i * LANES + lane_ids
        idx = jnp.where(pos < limit, idx, 0)
        return acc + plsc.load_gather(rs_v, [idx])

    acc_v[...] = lax.fori_loop(
        0, STEPS, body, jnp.zeros((LANES,), jnp.float32), unroll=8
    )
    pltpu.sync_copy(acc_v, shared.at[wid])
    plsc.subcore_barrier()

    @pl.when(wid == 0)
    def _():
        pltpu.sync_copy(shared, red_v)
        tot = red_v[0]
        for j in range(1, B):
            tot = tot + red_v[j]
        acc_v[...] = jnp.full((LANES,), lax.reduce_sum(tot, (0,)))
        pltpu.sync_copy(acc_v.at[pl.ds(0, 1)], out_hbm)


def kernel(tokens, seq_lengths, embed_table):
    row_sums = _row_sums(embed_table).reshape(VOCAB)
    total = _sc_gather_sum(row_sums, tokens.reshape(-1), seq_lengths)
    return total.reshape(())
